# baseline (speedup 1.0000x reference)
import functools

import jax
import jax.numpy as jnp
from jax import lax
from jax.experimental import pallas as pl
from jax.experimental.pallas import tpu as pltpu
from jax.experimental.pallas import tpu_sc as plsc

BATCH = 16384
EMB_DIM = 64
LANES = 128
NC = 2
NS = 16
NW = NC * NS
B_PER_W = BATCH // NW
NBUF = 8
CHUNK = 16
HALF = B_PER_W // 2
HALF_L = LANES // 2

SLOPE = 0.01
OUT_PAD = 128


def _fetch(tableT_hbm, idx_scalar, buf, sem):
    col0 = pl.multiple_of((idx_scalar >> 7) * LANES, LANES)
    pltpu.async_copy(tableT_hbm.at[:, pl.ds(col0, LANES)], buf, sem)


def _extract(buf, idx_scalar, rows_v, j):
    lane = jnp.broadcast_to(idx_scalar & (LANES - 1), (16,))
    for k in range(EMB_DIM // 16):
        c = lax.iota(jnp.int32, 16) + (16 * k)
        v = plsc.load_gather(buf, [c, lane])
        rows_v[j, pl.ds(16 * k, 16)] = v


def _sc_gather(user, tableT):
    n = user.shape[0]
    b_per_w = n // NW
    half = b_per_w // 2

    def body(idx_hbm, tableT_hbm, out_hbm, idx_v, bufs, rows_v, *sems):
        wid = lax.axis_index("s") * NC + lax.axis_index("c")
        base = wid * b_per_w
        pltpu.sync_copy(idx_hbm.at[wid], idx_v)

        v0 = idx_v[pl.ds(0, CHUNK)]
        for i in range(NBUF):
            _fetch(tableT_hbm, v0[i], bufs.at[i], sems[i])

        def chunk(c, carry):
            s0 = c * CHUNK
            v = idx_v[pl.ds(s0, CHUNK)]
            vn = idx_v[pl.ds(jnp.minimum(s0 + CHUNK, b_per_w - CHUNK),
                             CHUNK)]
            for i in range(CHUNK):
                s = s0 + i
                b = i % NBUF
                pltpu.make_async_copy(
                    tableT_hbm.at[:, pl.ds(0, LANES)], bufs.at[b],
                    sems[b]).wait()
                _extract(bufs.at[b], v[i], rows_v, s & (half - 1))
                nxt = v[i + NBUF] if i + NBUF < CHUNK else vn[i + NBUF - CHUNK]

                @pl.when(s + NBUF < b_per_w)
                def _():
                    _fetch(tableT_hbm, nxt, bufs.at[b], sems[b])

            @pl.when(c == (half // CHUNK) - 1)
            def _():
                pltpu.sync_copy(rows_v, out_hbm.at[pl.ds(base, half)])

            @pl.when(c == (b_per_w // CHUNK) - 1)
            def _():
                pltpu.sync_copy(rows_v, out_hbm.at[pl.ds(base + half, half)])

            return carry

        lax.fori_loop(0, b_per_w // CHUNK, chunk, 0, unroll=False)

    mesh = plsc.VectorSubcoreMesh(core_axis_name="c", subcore_axis_name="s")
    k = functools.partial(
        pl.kernel,
        mesh=mesh,
        out_type=jax.ShapeDtypeStruct((n, EMB_DIM), jnp.float32),
        scratch_types=[
            pltpu.VMEM((b_per_w,), jnp.int32),
            pltpu.VMEM((NBUF, EMB_DIM, LANES), jnp.float32),
            pltpu.VMEM((half, EMB_DIM), jnp.float32),
        ] + [pltpu.SemaphoreType.DMA] * NBUF,
        compiler_params=pltpu.CompilerParams(needs_layout_passes=False),
    )(body)
    return k(user.reshape(NW, b_per_w), tableT)


def _dot_t(a, w):
    return lax.dot_general(a, w, (((0,), (1,)), ((), ())),
                           preferred_element_type=jnp.float32)


def _dot_nt(a, w):
    return lax.dot_general(a, w, (((1,), (1,)), ((), ())),
                           preferred_element_type=jnp.float32)


def _mlp_body(g_ref, st_ref, w1_ref, b1_ref, w2_ref, b2_ref,
              w3_ref, b3_ref, w4_ref, b4_ref, o_ref):
    h = _dot_nt(g_ref[...], w1_ref[:, :EMB_DIM])
    h = h + _dot_t(st_ref[...], w1_ref[:, EMB_DIM:])
    h = h + b1_ref[...]
    h = jnp.where(h >= 0, h, SLOPE * h)
    h = _dot_nt(h, w2_ref[...]) + b2_ref[...]
    h = jnp.where(h >= 0, h, SLOPE * h)
    h = _dot_nt(h, w3_ref[...]) + b3_ref[...]
    h = jnp.where(h >= 0, h, SLOPE * h)
    o_ref[...] = _dot_nt(h, w4_ref[...]) + b4_ref[...]


MLP_BLK = 4096


def _mlp(g, st, w1, b1, w2, b2, w3, b3, w4p, b4p, col_off=0):
    n = g.shape[0]
    grid = (n // MLP_BLK,)
    const = lambda i: (0, 0)
    return pl.pallas_call(
        _mlp_body,
        grid=grid,
        in_specs=[
            pl.BlockSpec((MLP_BLK, EMB_DIM), lambda i: (i, 0)),
            pl.BlockSpec((EMB_DIM, MLP_BLK), lambda i: (0, i + col_off)),
            pl.BlockSpec(w1.shape, const),
            pl.BlockSpec(b1.shape, const),
            pl.BlockSpec(w2.shape, const),
            pl.BlockSpec(b2.shape, const),
            pl.BlockSpec(w3.shape, const),
            pl.BlockSpec(b3.shape, const),
            pl.BlockSpec(w4p.shape, const),
            pl.BlockSpec(b4p.shape, const),
        ],
        out_specs=pl.BlockSpec((MLP_BLK, OUT_PAD), lambda i: (i, 0)),
        out_shape=jax.ShapeDtypeStruct((n, OUT_PAD), jnp.float32),
        compiler_params=pltpu.CompilerParams(
            dimension_semantics=("arbitrary",),
        ),
    )(g, st, w1, b1, w2, b2, w3, b3, w4p, b4p)


NSPLIT = 4


def kernel(user, score, emb, W1, b1, W2, b2, W3, b3, W4, b4):
    embT = emb.T
    scoreT = score.T
    w4p = jnp.pad(W4, ((0, OUT_PAD - 5), (0, 0)))
    b4p = jnp.pad(b4, (0, OUT_PAD - 5))
    ph = BATCH // NSPLIT
    gs = [_sc_gather(user[k * ph:(k + 1) * ph], embT)
          for k in range(NSPLIT)]
    outs = [_mlp(gs[k], scoreT,
                 W1, b1.reshape(1, -1),
                 W2, b2.reshape(1, -1),
                 W3, b3.reshape(1, -1),
                 w4p, b4p.reshape(1, -1),
                 col_off=k * (ph // MLP_BLK))[:, :5]
            for k in range(NSPLIT)]
    return jnp.concatenate(outs, axis=0)


# speedup vs baseline: 1.0222x; 1.0222x over previous
import functools

import jax
import jax.numpy as jnp
from jax import lax
from jax.experimental import pallas as pl
from jax.experimental.pallas import tpu as pltpu
from jax.experimental.pallas import tpu_sc as plsc

BATCH = 16384
EMB_DIM = 64
LANES = 128
NC = 2
NS = 16
NW = NC * NS
B_PER_W = BATCH // NW
NBUF = 8
CHUNK = 16
HALF = B_PER_W // 2
HALF_L = LANES // 2

SLOPE = 0.01
OUT_PAD = 128


def _fetch(tableT_hbm, idx_scalar, buf, sem):
    col0 = pl.multiple_of((idx_scalar >> 7) * LANES, LANES)
    pltpu.async_copy(tableT_hbm.at[:, pl.ds(col0, LANES)], buf, sem)


def _extract(buf, idx_scalar, rows_v, j):
    lane = jnp.broadcast_to(idx_scalar & (LANES - 1), (16,))
    for k in range(EMB_DIM // 16):
        c = lax.iota(jnp.int32, 16) + (16 * k)
        v = plsc.load_gather(buf, [c, lane])
        rows_v[j, pl.ds(16 * k, 16)] = v


def _sc_gather(user, tableT):
    n = user.shape[0]
    b_per_w = n // NW
    half = b_per_w // 2

    def body(idx_hbm, tableT_hbm, out_hbm, idx_v, bufs, rows_v, *sems):
        wid = lax.axis_index("s") * NC + lax.axis_index("c")
        base = wid * b_per_w
        pltpu.sync_copy(idx_hbm.at[wid], idx_v)

        v0 = idx_v[pl.ds(0, CHUNK)]
        for i in range(NBUF):
            _fetch(tableT_hbm, v0[i], bufs.at[i], sems[i])

        def chunk(c, carry):
            s0 = c * CHUNK
            v = idx_v[pl.ds(s0, CHUNK)]
            vn = idx_v[pl.ds(jnp.minimum(s0 + CHUNK, b_per_w - CHUNK),
                             CHUNK)]
            for i in range(CHUNK):
                s = s0 + i
                b = i % NBUF
                pltpu.make_async_copy(
                    tableT_hbm.at[:, pl.ds(0, LANES)], bufs.at[b],
                    sems[b]).wait()
                _extract(bufs.at[b], v[i], rows_v, s & (half - 1))
                nxt = v[i + NBUF] if i + NBUF < CHUNK else vn[i + NBUF - CHUNK]

                @pl.when(s + NBUF < b_per_w)
                def _():
                    _fetch(tableT_hbm, nxt, bufs.at[b], sems[b])

            @pl.when(c == (half // CHUNK) - 1)
            def _():
                pltpu.sync_copy(rows_v, out_hbm.at[pl.ds(base, half)])

            @pl.when(c == (b_per_w // CHUNK) - 1)
            def _():
                pltpu.sync_copy(rows_v, out_hbm.at[pl.ds(base + half, half)])

            return carry

        lax.fori_loop(0, b_per_w // CHUNK, chunk, 0, unroll=False)

    mesh = plsc.VectorSubcoreMesh(core_axis_name="c", subcore_axis_name="s")
    k = functools.partial(
        pl.kernel,
        mesh=mesh,
        out_type=jax.ShapeDtypeStruct((n, EMB_DIM), jnp.float32),
        scratch_types=[
            pltpu.VMEM((b_per_w,), jnp.int32),
            pltpu.VMEM((NBUF, EMB_DIM, LANES), jnp.float32),
            pltpu.VMEM((half, EMB_DIM), jnp.float32),
        ] + [pltpu.SemaphoreType.DMA] * NBUF,
        compiler_params=pltpu.CompilerParams(needs_layout_passes=False),
    )(body)
    return k(user.reshape(NW, b_per_w), tableT)


def _dot_t(a, w):
    return lax.dot_general(a, w, (((0,), (1,)), ((), ())),
                           preferred_element_type=jnp.float32)


def _dot_nt(a, w):
    return lax.dot_general(a, w, (((1,), (1,)), ((), ())),
                           preferred_element_type=jnp.float32)


def _mlp_body(g_ref, st_ref, w1_ref, b1_ref, w2_ref, b2_ref,
              w3_ref, b3_ref, w4_ref, b4_ref, o_ref):
    h = _dot_nt(g_ref[...], w1_ref[:, :EMB_DIM])
    h = h + _dot_t(st_ref[...], w1_ref[:, EMB_DIM:])
    h = h + b1_ref[...]
    h = jnp.where(h >= 0, h, SLOPE * h)
    h = _dot_nt(h, w2_ref[...]) + b2_ref[...]
    h = jnp.where(h >= 0, h, SLOPE * h)
    h = _dot_nt(h, w3_ref[...]) + b3_ref[...]
    h = jnp.where(h >= 0, h, SLOPE * h)
    o_ref[...] = _dot_nt(h, w4_ref[...]) + b4_ref[...]


MLP_BLK = 4096


def _mlp(g, st, w1, b1, w2, b2, w3, b3, w4p, b4p, col_off=0):
    n = g.shape[0]
    grid = (n // MLP_BLK,)
    const = lambda i: (0, 0)
    return pl.pallas_call(
        _mlp_body,
        grid=grid,
        in_specs=[
            pl.BlockSpec((MLP_BLK, EMB_DIM), lambda i: (i, 0)),
            pl.BlockSpec((EMB_DIM, MLP_BLK), lambda i: (0, i + col_off)),
            pl.BlockSpec(w1.shape, const),
            pl.BlockSpec(b1.shape, const),
            pl.BlockSpec(w2.shape, const),
            pl.BlockSpec(b2.shape, const),
            pl.BlockSpec(w3.shape, const),
            pl.BlockSpec(b3.shape, const),
            pl.BlockSpec(w4p.shape, const),
            pl.BlockSpec(b4p.shape, const),
        ],
        out_specs=pl.BlockSpec((MLP_BLK, OUT_PAD), lambda i: (i, 0)),
        out_shape=jax.ShapeDtypeStruct((n, OUT_PAD), jnp.float32),
        compiler_params=pltpu.CompilerParams(
            dimension_semantics=("arbitrary",),
        ),
    )(g, st, w1, b1, w2, b2, w3, b3, w4p, b4p)


NSPLIT = 2


def kernel(user, score, emb, W1, b1, W2, b2, W3, b3, W4, b4):
    embT = emb.T
    scoreT = score.T
    w4p = jnp.pad(W4, ((0, OUT_PAD - 5), (0, 0)))
    b4p = jnp.pad(b4, (0, OUT_PAD - 5))
    ph = BATCH // NSPLIT
    gs = [_sc_gather(user[k * ph:(k + 1) * ph], embT)
          for k in range(NSPLIT)]
    outs = [_mlp(gs[k], scoreT,
                 W1, b1.reshape(1, -1),
                 W2, b2.reshape(1, -1),
                 W3, b3.reshape(1, -1),
                 w4p, b4p.reshape(1, -1),
                 col_off=k * (ph // MLP_BLK))[:, :5]
            for k in range(NSPLIT)]
    return jnp.concatenate(outs, axis=0)
